# Initial kernel scaffold; baseline (speedup 1.0000x reference)
import jax
import jax.numpy as jnp
from jax.experimental import pallas as pl


def kernel(X_input, tgt_emb_weight, pos_emb_weight):
    raise NotImplementedError("write your pallas kernel here")


# speedup vs baseline: 1.2353x; 1.2353x over previous
import jax
import jax.numpy as jnp
from jax import lax
from jax.experimental import pallas as pl
from jax.experimental.pallas import tpu as pltpu
from jax.experimental.pallas import tpu_sc as plsc

D = 768
NW = 32
SBLK = 64
CHUNK = 32
LPR = D // 16


def _emb_body(idx_hbm, tgt_hbm, pos_hbm, out_hbm, idx_v, pbuf, gbuf,
              sem_g, sem_o):
    n = out_hbm.shape[0]
    seq = pos_hbm.shape[0]
    batch = n // seq
    nch = batch * (SBLK // CHUNK)
    wid = lax.axis_index("s") * 2 + lax.axis_index("c")
    sbase = wid * SBLK

    pltpu.sync_copy(pos_hbm.at[pl.ds(sbase, SBLK)], pbuf)
    for b in range(batch):
        pltpu.sync_copy(idx_hbm.at[pl.ds(b * seq + sbase, SBLK)],
                        idx_v.at[pl.ds(b * SBLK, SBLK)])

    def out_slice(c):
        b, h = c // (SBLK // CHUNK), c % (SBLK // CHUNK)
        return pl.ds(b * seq + sbase + h * CHUNK, CHUNK)

    def start_gather(c):
        return pltpu.async_copy(
            tgt_hbm.at[idx_v.at[pl.ds(c * CHUNK, CHUNK)]],
            gbuf.at[c % 2], sem_g)

    descs_g = [None] * nch
    descs_o = [None] * nch
    descs_g[0] = start_gather(0)
    for c in range(nch):
        bb = c % 2
        if c + 1 < nch:
            if c >= 1:
                descs_o[c - 1].wait()
            descs_g[c + 1] = start_gather(c + 1)
        descs_g[c].wait()
        h = c % (SBLK // CHUNK)

        def add_row(r, _):
            for j in range(LPR):
                d = pl.ds(j * 16, 16)
                gbuf[bb, r, d] = gbuf[bb, r, d] + pbuf[h * CHUNK + r, d]
            return 0

        lax.fori_loop(0, CHUNK, add_row, 0)
        descs_o[c] = pltpu.async_copy(gbuf.at[bb], out_hbm.at[out_slice(c)],
                                      sem_o)
    descs_o[nch - 2].wait()
    descs_o[nch - 1].wait()


def kernel(X_input, tgt_emb_weight, pos_emb_weight):
    batch, seq = X_input.shape
    n = batch * seq
    idx = X_input.reshape(n).astype(jnp.int32)

    mesh = plsc.VectorSubcoreMesh(core_axis_name="c", subcore_axis_name="s")
    run = pl.kernel(
        _emb_body,
        out_type=jax.ShapeDtypeStruct((n, D), jnp.float32),
        mesh=mesh,
        scratch_types=[
            pltpu.VMEM((batch * SBLK,), jnp.int32),
            pltpu.VMEM((SBLK, D), jnp.float32),
            pltpu.VMEM((2, CHUNK, D), jnp.float32),
            pltpu.SemaphoreType.DMA,
            pltpu.SemaphoreType.DMA,
        ],
    )
    out = run(idx, tgt_emb_weight, pos_emb_weight)
    return out.reshape(batch, seq, D)


# speedup vs baseline: 1.2477x; 1.0101x over previous
import jax
import jax.numpy as jnp
from jax import lax
from jax.experimental import pallas as pl
from jax.experimental.pallas import tpu as pltpu
from jax.experimental.pallas import tpu_sc as plsc

D = 768
NW = 32
SBLK = 64
CHUNK = 16
RING = 6
LPR = D // 16


def _emb_body(idx_hbm, tgt_hbm, pos_hbm, out_hbm, idx_v, pbuf, gbuf,
              sem_p, sem_g, sem_o):
    n = out_hbm.shape[0]
    seq = pos_hbm.shape[0]
    batch = n // seq
    cpb = SBLK // CHUNK
    nch = batch * cpb
    wid = lax.axis_index("s") * 2 + lax.axis_index("c")
    sbase = wid * SBLK

    dp = pltpu.async_copy(pos_hbm.at[pl.ds(sbase, SBLK)], pbuf, sem_p)
    for b in range(batch):
        pltpu.sync_copy(idx_hbm.at[pl.ds(b * seq + sbase, SBLK)],
                        idx_v.at[pl.ds(b * SBLK, SBLK)])

    def out_slice(c):
        b, h = c // cpb, c % cpb
        return pl.ds(b * seq + sbase + h * CHUNK, CHUNK)

    def start_gather(c):
        return pltpu.async_copy(
            tgt_hbm.at[idx_v.at[pl.ds(c * CHUNK, CHUNK)]],
            gbuf.at[c % RING], sem_g)

    descs_g = [None] * nch
    descs_o = [None] * nch
    for c in range(RING - 1):
        descs_g[c] = start_gather(c)
    dp.wait()
    for c in range(nch):
        bb = c % RING
        descs_g[c].wait()
        h = c % cpb

        def add_row(r, _):
            for j in range(LPR):
                d = pl.ds(j * 16, 16)
                gbuf[bb, r, d] = gbuf[bb, r, d] + pbuf[h * CHUNK + r, d]
            return 0

        lax.fori_loop(0, CHUNK, add_row, 0)
        descs_o[c] = pltpu.async_copy(gbuf.at[bb], out_hbm.at[out_slice(c)],
                                      sem_o)
        if c + RING - 1 < nch:
            if c >= 1:
                descs_o[c - 1].wait()
            descs_g[c + RING - 1] = start_gather(c + RING - 1)
    for c in range(max(0, nch - RING), nch):
        descs_o[c].wait()


def kernel(X_input, tgt_emb_weight, pos_emb_weight):
    batch, seq = X_input.shape
    n = batch * seq
    idx = X_input.reshape(n).astype(jnp.int32)

    mesh = plsc.VectorSubcoreMesh(core_axis_name="c", subcore_axis_name="s")
    run = pl.kernel(
        _emb_body,
        out_type=jax.ShapeDtypeStruct((n, D), jnp.float32),
        mesh=mesh,
        scratch_types=[
            pltpu.VMEM((batch * SBLK,), jnp.int32),
            pltpu.VMEM((SBLK, D), jnp.float32),
            pltpu.VMEM((RING, CHUNK, D), jnp.float32),
            pltpu.SemaphoreType.DMA,
            pltpu.SemaphoreType.DMA,
            pltpu.SemaphoreType.DMA,
        ],
    )
    out = run(idx, tgt_emb_weight, pos_emb_weight)
    return out.reshape(batch, seq, D)
